# baseline (speedup 1.0000x reference)
import functools

import jax
import jax.numpy as jnp
from jax.experimental import pallas as pl
from jax.experimental.pallas import tpu as pltpu

_MMDT = jnp.bfloat16
_EPS = 1e-6
_GROUPS = 32


def _gn_swish(x, grp_ref, gamma_ref, beta_ref, inv_n):
    s = jnp.sum(x, axis=0, keepdims=True)
    s2 = jnp.sum(x * x, axis=0, keepdims=True)
    gs = jnp.dot(s, grp_ref[...], preferred_element_type=jnp.float32)
    gs2 = jnp.dot(s2, grp_ref[...], preferred_element_type=jnp.float32)
    mean = gs * inv_n
    var = jnp.maximum(gs2 * inv_n - mean * mean, 0.0)
    a = jax.lax.rsqrt(var + _EPS) * gamma_ref[...]
    bb = beta_ref[...] - mean * a
    y = x * a + bb
    return y * jax.nn.sigmoid(y)


def _fill_pad(pad_ref, y_hwc, H, W):
    C = pad_ref.shape[2]
    zrow = jnp.zeros((1, W + 2, C), pad_ref.dtype)
    zcol = jnp.zeros((H + 2, 1, C), pad_ref.dtype)
    pad_ref[0:1] = zrow
    pad_ref[H + 1:H + 2] = zrow
    pad_ref[:, 0:1, :] = zcol
    pad_ref[:, W + 1:W + 2, :] = zcol
    pad_ref[1:H + 1, 1:W + 1, :] = y_hwc.astype(pad_ref.dtype)


def _conv_from_pad(pad_ref, w_ref, H, W, C, Cout):
    acc = None
    for dx in range(3):
        xs = pad_ref[:, dx:dx + W, :].reshape((H + 2) * W, C)
        for dy in range(3):
            t = jnp.dot(xs[dy * W:dy * W + H * W], w_ref[3 * dy + dx],
                        preferred_element_type=jnp.float32)
            acc = t if acc is None else acc + t
    return acc


def _dec_body(x_ref, win_ref, bin_ref,
              g01_ref, b01_ref, w01_ref, g11_ref, b11_ref, w11_ref,
              g02_ref, b02_ref, w02_ref, g12_ref, b12_ref, w12_ref,
              wup_ref, bup_ref,
              g03_ref, b03_ref, w03_ref, g13_ref, b13_ref, w13_ref, wr_ref,
              ggn_ref, bgn_ref, wout_ref, bout_ref,
              grpa_ref, grpb_ref,
              o_ref, pa_ref, pc_ref, pd_ref, *, H0, W0, C2, C1, Cout):
    H1, W1 = 2 * H0, 2 * W0
    inv_a = 1.0 / float(H0 * W0 * (C2 // _GROUPS))
    inv_c = 1.0 / float(H1 * W1 * (C2 // _GROUPS))
    inv_d = 1.0 / float(H1 * W1 * (C1 // _GROUPS))

    x = x_ref[0].reshape(H0 * W0, x_ref.shape[3]).astype(jnp.float32)
    _fill_pad(pa_ref, x.reshape(H0, W0, x_ref.shape[3]), H0, W0)
    h = _conv_from_pad(pa_ref, win_ref, H0, W0, x_ref.shape[3],
                       C2) + bin_ref[...]
    for g0, b0, w0, g1, b1, w1 in (
            (g01_ref, b01_ref, w01_ref, g11_ref, b11_ref, w11_ref),
            (g02_ref, b02_ref, w02_ref, g12_ref, b12_ref, w12_ref)):
        y = _gn_swish(h, grpa_ref, g0, b0, inv_a)
        _fill_pad(pa_ref, y.reshape(H0, W0, C2), H0, W0)
        t = _conv_from_pad(pa_ref, w0, H0, W0, C2, C2)
        y = _gn_swish(t, grpa_ref, g1, b1, inv_a)
        _fill_pad(pa_ref, y.reshape(H0, W0, C2), H0, W0)
        h = _conv_from_pad(pa_ref, w1, H0, W0, C2, C2) + h
    u = h.reshape(H0, W0, C2)
    uw = jnp.concatenate([u[:, :, None, :], u[:, :, None, :]],
                         axis=2).reshape(H0, W1, C2)
    uh = jnp.concatenate([uw[:, None], uw[:, None]],
                         axis=1).reshape(H1, W1, C2)
    _fill_pad(pc_ref, uh, H1, W1)
    h = _conv_from_pad(pc_ref, wup_ref, H1, W1, C2, C2) + bup_ref[...]
    y = _gn_swish(h, grpa_ref, g03_ref, b03_ref, inv_c)
    _fill_pad(pc_ref, y.reshape(H1, W1, C2), H1, W1)
    t = _conv_from_pad(pc_ref, w03_ref, H1, W1, C2, C1)
    y = _gn_swish(t, grpb_ref, g13_ref, b13_ref, inv_d)
    _fill_pad(pd_ref, y.reshape(H1, W1, C1), H1, W1)
    t2 = _conv_from_pad(pd_ref, w13_ref, H1, W1, C1, C1)
    h = t2 + jnp.dot(t2.astype(_MMDT), wr_ref[...],
                     preferred_element_type=jnp.float32)
    y = _gn_swish(h, grpb_ref, ggn_ref, bgn_ref, inv_d)
    _fill_pad(pd_ref, y.reshape(H1, W1, C1), H1, W1)
    out = _conv_from_pad(pd_ref, wout_ref, H1, W1, C1, Cout) + bout_ref[...]
    o_ref[0] = out.reshape(H1, W1, Cout)


def _grp_matrix(C):
    gid = jnp.arange(C, dtype=jnp.int32) // (C // _GROUPS)
    return (gid[:, None] == gid[None, :]).astype(jnp.float32)


def _wmat3x3(w):
    Cout, Cin = w.shape[0], w.shape[1]
    return jnp.transpose(w, (2, 3, 1, 0)).reshape(9, Cin, Cout).astype(_MMDT)


def _row(v, C):
    return v.reshape(1, C).astype(jnp.float32)


def kernel(x, p00, p01, p02, p03, p04, p05, p06, p07, p08, p09, p10, p11,
           p12, p13, p14, p15, p16, p17, p18, p19, p20, p21, p22, p23, p24,
           p25, p26):
    B, Cemb, H0, W0 = x.shape
    C2 = p01.shape[0]
    C1 = p21.shape[0]
    Cout = p03.shape[0]
    H1, W1 = 2 * H0, 2 * W0
    xh = jnp.transpose(x, (0, 2, 3, 1)).astype(jnp.float32)

    args = [
        xh,
        _wmat3x3(p01), _row(p00, C2),
        _row(p09, C2), _row(p08, C2), _wmat3x3(p06),
        _row(p11, C2), _row(p10, C2), _wmat3x3(p07),
        _row(p15, C2), _row(p14, C2), _wmat3x3(p12),
        _row(p17, C2), _row(p16, C2), _wmat3x3(p13),
        _wmat3x3(p19), _row(p18, C2),
        _row(p24, C2), _row(p23, C2), _wmat3x3(p20),
        _row(p26, C1), _row(p25, C1), _wmat3x3(p21),
        jnp.transpose(p22.reshape(C1, C1)).astype(_MMDT),
        _row(p05, C1), _row(p04, C1), _wmat3x3(p03), _row(p02, Cout),
        _grp_matrix(C2), _grp_matrix(C1),
    ]
    resident = lambda *shape: pl.BlockSpec(shape, lambda i: (0,) * len(shape))
    in_specs = [pl.BlockSpec((1, H0, W0, Cemb), lambda i: (i, 0, 0, 0))]
    for a in args[1:]:
        in_specs.append(resident(*a.shape))

    body = functools.partial(_dec_body, H0=H0, W0=W0, C2=C2, C1=C1, Cout=Cout)
    out = pl.pallas_call(
        body,
        out_shape=jax.ShapeDtypeStruct((B, H1, W1, Cout), jnp.float32),
        grid_spec=pltpu.PrefetchScalarGridSpec(
            num_scalar_prefetch=0,
            grid=(B,),
            in_specs=in_specs,
            out_specs=pl.BlockSpec((1, H1, W1, Cout), lambda i: (i, 0, 0, 0)),
            scratch_shapes=[
                pltpu.VMEM((H0 + 2, W0 + 2, max(Cemb, C2)), _MMDT),
                pltpu.VMEM((H1 + 2, W1 + 2, C2), _MMDT),
                pltpu.VMEM((H1 + 2, W1 + 2, C1), _MMDT)],
        ),
        compiler_params=pltpu.CompilerParams(
            dimension_semantics=("parallel",)),
    )(*args)
    return jnp.transpose(out, (0, 3, 1, 2))


# speedup vs baseline: 1.3296x; 1.3296x over previous
import functools

import jax
import jax.numpy as jnp
from jax.experimental import pallas as pl
from jax.experimental.pallas import tpu as pltpu

_MMDT = jnp.bfloat16
_EPS = 1e-6
_GROUPS = 32


def _gn_swish(x, grp_ref, gamma_ref, beta_ref, inv_n):
    s = jnp.sum(x, axis=0, keepdims=True)
    s2 = jnp.sum(x * x, axis=0, keepdims=True)
    gs = jnp.dot(s, grp_ref[...], preferred_element_type=jnp.float32)
    gs2 = jnp.dot(s2, grp_ref[...], preferred_element_type=jnp.float32)
    mean = gs * inv_n
    var = jnp.maximum(gs2 * inv_n - mean * mean, 0.0)
    a = jax.lax.rsqrt(var + _EPS) * gamma_ref[...]
    bb = beta_ref[...] - mean * a
    y = x * a + bb
    return y * jax.nn.sigmoid(y)


def _fill_pad(pad_ref, y_hwc, H, W):
    C = pad_ref.shape[2]
    zrow = jnp.zeros((1, W + 2, C), pad_ref.dtype)
    zcol = jnp.zeros((H + 2, 1, C), pad_ref.dtype)
    pad_ref[0:1] = zrow
    pad_ref[H + 1:H + 2] = zrow
    pad_ref[:, 0:1, :] = zcol
    pad_ref[:, W + 1:W + 2, :] = zcol
    pad_ref[1:H + 1, 1:W + 1, :] = y_hwc.astype(pad_ref.dtype)


def _conv_from_pad(pad_ref, w_ref, H, W, C, Cout):
    acc = None
    for dx in range(3):
        xs = pad_ref[:, dx:dx + W, :].reshape((H + 2) * W, C)
        for dy in range(3):
            t = jnp.dot(xs[dy * W:dy * W + H * W], w_ref[3 * dy + dx],
                        preferred_element_type=jnp.float32)
            acc = t if acc is None else acc + t
    return acc


def _stage_a_body(x_ref, win_ref, bin_ref,
                  g01_ref, b01_ref, w01_ref, g11_ref, b11_ref, w11_ref,
                  g02_ref, b02_ref, w02_ref, g12_ref, b12_ref, w12_ref,
                  grpa_ref, o_ref, pa_ref, *, H0, W0, C2):
    Cemb = x_ref.shape[3]
    inv_a = 1.0 / float(H0 * W0 * (C2 // _GROUPS))
    x = x_ref[0].reshape(H0 * W0, Cemb).astype(jnp.float32)
    _fill_pad(pa_ref, x.reshape(H0, W0, Cemb), H0, W0)
    h = _conv_from_pad(pa_ref, win_ref, H0, W0, Cemb, C2) + bin_ref[...]
    for g0, b0, w0, g1, b1, w1 in (
            (g01_ref, b01_ref, w01_ref, g11_ref, b11_ref, w11_ref),
            (g02_ref, b02_ref, w02_ref, g12_ref, b12_ref, w12_ref)):
        y = _gn_swish(h, grpa_ref, g0, b0, inv_a)
        _fill_pad(pa_ref, y.reshape(H0, W0, C2), H0, W0)
        t = _conv_from_pad(pa_ref, w0, H0, W0, C2, C2)
        y = _gn_swish(t, grpa_ref, g1, b1, inv_a)
        _fill_pad(pa_ref, y.reshape(H0, W0, C2), H0, W0)
        h = _conv_from_pad(pa_ref, w1, H0, W0, C2, C2) + h
    o_ref[0] = h.reshape(H0, W0, C2)


def _upconv_body(x_ref, w_ref, b_ref, o_ref, pad_ref, *, H, W, C):
    x = x_ref[0]
    xw = jnp.concatenate([x[:, :, None, :], x[:, :, None, :]],
                         axis=2).reshape(H, 2 * W, C)
    xh = jnp.concatenate([xw[:, None], xw[:, None]],
                         axis=1).reshape(2 * H, 2 * W, C)
    _fill_pad(pad_ref, xh, 2 * H, 2 * W)
    acc = _conv_from_pad(pad_ref, w_ref, 2 * H, 2 * W, C, C) + b_ref[...]
    o_ref[0] = acc.reshape(2 * H, 2 * W, C)


def _stage_c_body(x_ref, g03_ref, b03_ref, w03_ref, g13_ref, b13_ref,
                  w13_ref, wr_ref, ggn_ref, bgn_ref, wout_ref, bout_ref,
                  grpa_ref, grpb_ref, o_ref, pc_ref, pd_ref,
                  *, H1, W1, C2, C1, Cout):
    inv_c = 1.0 / float(H1 * W1 * (C2 // _GROUPS))
    inv_d = 1.0 / float(H1 * W1 * (C1 // _GROUPS))
    h = x_ref[0].reshape(H1 * W1, C2).astype(jnp.float32)
    y = _gn_swish(h, grpa_ref, g03_ref, b03_ref, inv_c)
    _fill_pad(pc_ref, y.reshape(H1, W1, C2), H1, W1)
    t = _conv_from_pad(pc_ref, w03_ref, H1, W1, C2, C1)
    y = _gn_swish(t, grpb_ref, g13_ref, b13_ref, inv_d)
    _fill_pad(pd_ref, y.reshape(H1, W1, C1), H1, W1)
    t2 = _conv_from_pad(pd_ref, w13_ref, H1, W1, C1, C1)
    h = t2 + jnp.dot(t2.astype(_MMDT), wr_ref[...],
                     preferred_element_type=jnp.float32)
    y = _gn_swish(h, grpb_ref, ggn_ref, bgn_ref, inv_d)
    _fill_pad(pd_ref, y.reshape(H1, W1, C1), H1, W1)
    out = _conv_from_pad(pd_ref, wout_ref, H1, W1, C1, Cout) + bout_ref[...]
    o_ref[0] = out.reshape(H1, W1, Cout)


def _grp_matrix(C):
    gid = jnp.arange(C, dtype=jnp.int32) // (C // _GROUPS)
    return (gid[:, None] == gid[None, :]).astype(jnp.float32)


def _wmat3x3(w):
    Cout, Cin = w.shape[0], w.shape[1]
    return jnp.transpose(w, (2, 3, 1, 0)).reshape(9, Cin, Cout).astype(_MMDT)


def _row(v, C):
    return v.reshape(1, C).astype(jnp.float32)


def kernel(x, p00, p01, p02, p03, p04, p05, p06, p07, p08, p09, p10, p11,
           p12, p13, p14, p15, p16, p17, p18, p19, p20, p21, p22, p23, p24,
           p25, p26):
    B, Cemb, H0, W0 = x.shape
    C2 = p01.shape[0]
    C1 = p21.shape[0]
    Cout = p03.shape[0]
    H1, W1 = 2 * H0, 2 * W0
    xh = jnp.transpose(x, (0, 2, 3, 1)).astype(jnp.float32)
    resident = lambda *shape: pl.BlockSpec(shape, lambda i: (0,) * len(shape))
    par = pltpu.CompilerParams(dimension_semantics=("parallel",))

    a_args = [
        xh, _wmat3x3(p01), _row(p00, C2),
        _row(p09, C2), _row(p08, C2), _wmat3x3(p06),
        _row(p11, C2), _row(p10, C2), _wmat3x3(p07),
        _row(p15, C2), _row(p14, C2), _wmat3x3(p12),
        _row(p17, C2), _row(p16, C2), _wmat3x3(p13),
        _grp_matrix(C2),
    ]
    a_specs = [pl.BlockSpec((1, H0, W0, Cemb), lambda i: (i, 0, 0, 0))]
    a_specs += [resident(*a.shape) for a in a_args[1:]]
    h = pl.pallas_call(
        functools.partial(_stage_a_body, H0=H0, W0=W0, C2=C2),
        out_shape=jax.ShapeDtypeStruct((B, H0, W0, C2), jnp.float32),
        grid_spec=pltpu.PrefetchScalarGridSpec(
            num_scalar_prefetch=0, grid=(B,), in_specs=a_specs,
            out_specs=pl.BlockSpec((1, H0, W0, C2), lambda i: (i, 0, 0, 0)),
            scratch_shapes=[pltpu.VMEM((H0 + 2, W0 + 2, max(Cemb, C2)),
                                       _MMDT)],
        ),
        compiler_params=par,
    )(*a_args)

    h = pl.pallas_call(
        functools.partial(_upconv_body, H=H0, W=W0, C=C2),
        out_shape=jax.ShapeDtypeStruct((B, H1, W1, C2), jnp.float32),
        grid_spec=pltpu.PrefetchScalarGridSpec(
            num_scalar_prefetch=0, grid=(B,),
            in_specs=[pl.BlockSpec((1, H0, W0, C2), lambda i: (i, 0, 0, 0)),
                      resident(9, C2, C2), resident(1, C2)],
            out_specs=pl.BlockSpec((1, H1, W1, C2), lambda i: (i, 0, 0, 0)),
            scratch_shapes=[pltpu.VMEM((H1 + 2, W1 + 2, C2), _MMDT)],
        ),
        compiler_params=par,
    )(h, _wmat3x3(p19), _row(p18, C2))

    c_args = [
        h,
        _row(p24, C2), _row(p23, C2), _wmat3x3(p20),
        _row(p26, C1), _row(p25, C1), _wmat3x3(p21),
        jnp.transpose(p22.reshape(C1, C1)).astype(_MMDT),
        _row(p05, C1), _row(p04, C1), _wmat3x3(p03), _row(p02, Cout),
        _grp_matrix(C2), _grp_matrix(C1),
    ]
    c_specs = [pl.BlockSpec((1, H1, W1, C2), lambda i: (i, 0, 0, 0))]
    c_specs += [resident(*a.shape) for a in c_args[1:]]
    out = pl.pallas_call(
        functools.partial(_stage_c_body, H1=H1, W1=W1, C2=C2, C1=C1,
                          Cout=Cout),
        out_shape=jax.ShapeDtypeStruct((B, H1, W1, Cout), jnp.float32),
        grid_spec=pltpu.PrefetchScalarGridSpec(
            num_scalar_prefetch=0, grid=(B,), in_specs=c_specs,
            out_specs=pl.BlockSpec((1, H1, W1, Cout), lambda i: (i, 0, 0, 0)),
            scratch_shapes=[pltpu.VMEM((H1 + 2, W1 + 2, C2), _MMDT),
                            pltpu.VMEM((H1 + 2, W1 + 2, C1), _MMDT)],
        ),
        compiler_params=par,
    )(*c_args)
    return jnp.transpose(out, (0, 3, 1, 2))


# speedup vs baseline: 1.3603x; 1.0231x over previous
import functools

import jax
import jax.numpy as jnp
from jax.experimental import pallas as pl
from jax.experimental.pallas import tpu as pltpu

_MMDT = jnp.bfloat16
_EPS = 1e-6
_GROUPS = 32


def _gn_swish(x, grp_ref, gamma_ref, beta_ref, inv_n):
    s = jnp.sum(x, axis=0, keepdims=True)
    s2 = jnp.sum(x * x, axis=0, keepdims=True)
    gs = jnp.dot(s, grp_ref[...], preferred_element_type=jnp.float32)
    gs2 = jnp.dot(s2, grp_ref[...], preferred_element_type=jnp.float32)
    mean = gs * inv_n
    var = jnp.maximum(gs2 * inv_n - mean * mean, 0.0)
    a = jax.lax.rsqrt(var + _EPS) * gamma_ref[...]
    bb = beta_ref[...] - mean * a
    y = x * a + bb
    return y * jax.nn.sigmoid(y)


def _store_taps(x3_ref, y, H, W, C):
    HW = H * W
    yb = y.astype(x3_ref.dtype)
    col = jax.lax.broadcasted_iota(jnp.int32, (HW, 1), 0) % W
    m_l = (col < W - 1).astype(x3_ref.dtype)
    m_r = (col > 0).astype(x3_ref.dtype)
    for dx, ym in ((0, yb * m_l), (1, yb), (2, yb * m_r)):
        off = W + 1 - dx
        x3_ref[0:off, dx * C:(dx + 1) * C] = jnp.zeros((off, C), x3_ref.dtype)
        x3_ref[off:off + HW, dx * C:(dx + 1) * C] = ym
        tail = W - 1 + dx
        x3_ref[off + HW:off + HW + tail, dx * C:(dx + 1) * C] = (
            jnp.zeros((tail, C), x3_ref.dtype))


def _conv3(x3_ref, w_ref, H, W, Cout):
    HW = H * W
    acc = None
    for dy in range(3):
        t = jnp.dot(x3_ref[dy * W:dy * W + HW, :], w_ref[dy],
                    preferred_element_type=jnp.float32)
        acc = t if acc is None else acc + t
    return acc


def _gnconv_body(x_ref, gamma_ref, beta_ref, grp_ref, w_ref, b_ref, o_ref,
                 x3_ref, *, H, W, C, Cout, inv_n, use_gn):
    xm = x_ref[0].reshape(H * W, C).astype(jnp.float32)
    y = _gn_swish(xm, grp_ref, gamma_ref, beta_ref, inv_n) if use_gn else xm
    _store_taps(x3_ref, y, H, W, C)
    acc = _conv3(x3_ref, w_ref, H, W, Cout) + b_ref[...]
    o_ref[0] = acc.reshape(H, W, Cout)


def _res_body(x_ref, g0_ref, b0_ref, g1_ref, b1_ref, grp0_ref, grp1_ref,
              w0_ref, w1_ref, *rest, H, W, Cin, Cout, inv0, inv1, proj):
    if proj:
        wr_ref, o_ref, x3a_ref, x3b_ref = rest
    else:
        o_ref, x3a_ref, x3b_ref = rest
    xm = x_ref[0].reshape(H * W, Cin).astype(jnp.float32)
    y0 = _gn_swish(xm, grp0_ref, g0_ref, b0_ref, inv0)
    _store_taps(x3a_ref, y0, H, W, Cin)
    h = _conv3(x3a_ref, w0_ref, H, W, Cout)
    y1 = _gn_swish(h, grp1_ref, g1_ref, b1_ref, inv1)
    _store_taps(x3b_ref, y1, H, W, Cout)
    h2 = _conv3(x3b_ref, w1_ref, H, W, Cout)
    if proj:
        out = h2 + jnp.dot(h2.astype(_MMDT), wr_ref[...],
                           preferred_element_type=jnp.float32)
    else:
        out = h2 + xm
    o_ref[0] = out.reshape(H, W, Cout)


def _upconv_body(x_ref, w_ref, b_ref, o_ref, x3_ref, *, H, W, C):
    x = x_ref[0]
    xw = jnp.concatenate([x[:, :, None, :], x[:, :, None, :]],
                         axis=2).reshape(H, 2 * W, C)
    xh = jnp.concatenate([xw[:, None], xw[:, None]],
                         axis=1).reshape(2 * H, 2 * W, C)
    _store_taps(x3_ref, xh.reshape(4 * H * W, C), 2 * H, 2 * W, C)
    acc = _conv3(x3_ref, w_ref, 2 * H, 2 * W, C) + b_ref[...]
    o_ref[0] = acc.reshape(2 * H, 2 * W, C)


def _grp_matrix(C):
    gid = jnp.arange(C, dtype=jnp.int32) // (C // _GROUPS)
    return (gid[:, None] == gid[None, :]).astype(jnp.float32)


def _wmat3x3(w):
    Cout, Cin = w.shape[0], w.shape[1]
    return jnp.transpose(w, (2, 3, 1, 0)).reshape(3, 3 * Cin,
                                                  Cout).astype(_MMDT)


def _bias_row(b, Cout):
    return (jnp.zeros((1, Cout), jnp.float32) if b is None
            else b.reshape(1, Cout).astype(jnp.float32))


_PAR = pltpu.CompilerParams(dimension_semantics=("parallel",))


def _gn_conv(x, gamma, beta, w, b, *, use_gn):
    B, H, W, C = x.shape
    Cout = w.shape[0]
    if use_gn:
        gam = gamma.reshape(1, C).astype(jnp.float32)
        bet = beta.reshape(1, C).astype(jnp.float32)
    else:
        gam = jnp.ones((1, C), jnp.float32)
        bet = jnp.zeros((1, C), jnp.float32)
    body = functools.partial(_gnconv_body, H=H, W=W, C=C, Cout=Cout,
                             inv_n=1.0 / float(H * W * (C // _GROUPS)),
                             use_gn=use_gn)
    return pl.pallas_call(
        body,
        out_shape=jax.ShapeDtypeStruct((B, H, W, Cout), jnp.float32),
        grid_spec=pltpu.PrefetchScalarGridSpec(
            num_scalar_prefetch=0,
            grid=(B,),
            in_specs=[
                pl.BlockSpec((1, H, W, C), lambda i: (i, 0, 0, 0)),
                pl.BlockSpec((1, C), lambda i: (0, 0)),
                pl.BlockSpec((1, C), lambda i: (0, 0)),
                pl.BlockSpec((C, C), lambda i: (0, 0)),
                pl.BlockSpec((3, 3 * C, Cout), lambda i: (0, 0, 0)),
                pl.BlockSpec((1, Cout), lambda i: (0, 0)),
            ],
            out_specs=pl.BlockSpec((1, H, W, Cout), lambda i: (i, 0, 0, 0)),
            scratch_shapes=[pltpu.VMEM(((H + 2) * W, 3 * C), _MMDT)],
        ),
        compiler_params=_PAR,
    )(x, gam, bet, _grp_matrix(C), _wmat3x3(w), _bias_row(b, Cout))


def _res_block(x, g0, b0, w0, g1, b1, w1, wr):
    B, H, W, Cin = x.shape
    Cout = w0.shape[0]
    proj = wr is not None
    body = functools.partial(_res_body, H=H, W=W, Cin=Cin, Cout=Cout,
                             inv0=1.0 / float(H * W * (Cin // _GROUPS)),
                             inv1=1.0 / float(H * W * (Cout // _GROUPS)),
                             proj=proj)
    in_specs = [
        pl.BlockSpec((1, H, W, Cin), lambda i: (i, 0, 0, 0)),
        pl.BlockSpec((1, Cin), lambda i: (0, 0)),
        pl.BlockSpec((1, Cin), lambda i: (0, 0)),
        pl.BlockSpec((1, Cout), lambda i: (0, 0)),
        pl.BlockSpec((1, Cout), lambda i: (0, 0)),
        pl.BlockSpec((Cin, Cin), lambda i: (0, 0)),
        pl.BlockSpec((Cout, Cout), lambda i: (0, 0)),
        pl.BlockSpec((3, 3 * Cin, Cout), lambda i: (0, 0, 0)),
        pl.BlockSpec((3, 3 * Cout, Cout), lambda i: (0, 0, 0)),
    ]
    args = [x,
            g0.reshape(1, Cin).astype(jnp.float32),
            b0.reshape(1, Cin).astype(jnp.float32),
            g1.reshape(1, Cout).astype(jnp.float32),
            b1.reshape(1, Cout).astype(jnp.float32),
            _grp_matrix(Cin), _grp_matrix(Cout), _wmat3x3(w0), _wmat3x3(w1)]
    if proj:
        in_specs.append(pl.BlockSpec((Cout, Cout), lambda i: (0, 0)))
        args.append(jnp.transpose(wr.reshape(Cout, Cout)).astype(_MMDT))
    return pl.pallas_call(
        body,
        out_shape=jax.ShapeDtypeStruct((B, H, W, Cout), jnp.float32),
        grid_spec=pltpu.PrefetchScalarGridSpec(
            num_scalar_prefetch=0,
            grid=(B,),
            in_specs=in_specs,
            out_specs=pl.BlockSpec((1, H, W, Cout), lambda i: (i, 0, 0, 0)),
            scratch_shapes=[pltpu.VMEM(((H + 2) * W, 3 * Cin), _MMDT),
                            pltpu.VMEM(((H + 2) * W, 3 * Cout), _MMDT)],
        ),
        compiler_params=_PAR,
    )(*args)


def _up_conv(x, w, b):
    B, H, W, C = x.shape
    Cout = w.shape[0]
    body = functools.partial(_upconv_body, H=H, W=W, C=C)
    return pl.pallas_call(
        body,
        out_shape=jax.ShapeDtypeStruct((B, 2 * H, 2 * W, Cout), jnp.float32),
        grid_spec=pltpu.PrefetchScalarGridSpec(
            num_scalar_prefetch=0,
            grid=(B,),
            in_specs=[
                pl.BlockSpec((1, H, W, C), lambda i: (i, 0, 0, 0)),
                pl.BlockSpec((3, 3 * C, Cout), lambda i: (0, 0, 0)),
                pl.BlockSpec((1, Cout), lambda i: (0, 0)),
            ],
            out_specs=pl.BlockSpec((1, 2 * H, 2 * W, Cout),
                                   lambda i: (i, 0, 0, 0)),
            scratch_shapes=[pltpu.VMEM(((2 * H + 2) * 2 * W, 3 * C), _MMDT)],
        ),
        compiler_params=_PAR,
    )(x, _wmat3x3(w), _bias_row(b, Cout))


def kernel(x, p00, p01, p02, p03, p04, p05, p06, p07, p08, p09, p10, p11,
           p12, p13, p14, p15, p16, p17, p18, p19, p20, p21, p22, p23, p24,
           p25, p26):
    h = jnp.transpose(x, (0, 2, 3, 1)).astype(jnp.float32)
    h = _gn_conv(h, None, None, p01, p00, use_gn=False)
    h = _res_block(h, p09, p08, p06, p11, p10, p07, None)
    h = _res_block(h, p15, p14, p12, p17, p16, p13, None)
    h = _up_conv(h, p19, p18)
    h = _res_block(h, p24, p23, p20, p26, p25, p21, p22)
    h = _gn_conv(h, p05, p04, p03, p02, use_gn=True)
    return jnp.transpose(h, (0, 3, 1, 2))


# speedup vs baseline: 1.4211x; 1.0447x over previous
import functools

import jax
import jax.numpy as jnp
from jax.experimental import pallas as pl
from jax.experimental.pallas import tpu as pltpu

_MMDT = jnp.bfloat16
_EPS = 1e-6
_GROUPS = 32


def _gn_swish(x, grp_ref, gamma_ref, beta_ref, inv_n):
    s = jnp.sum(x, axis=0, keepdims=True)
    s2 = jnp.sum(x * x, axis=0, keepdims=True)
    gs = jnp.dot(s, grp_ref[...], preferred_element_type=jnp.float32)
    gs2 = jnp.dot(s2, grp_ref[...], preferred_element_type=jnp.float32)
    mean = gs * inv_n
    var = jnp.maximum(gs2 * inv_n - mean * mean, 0.0)
    a = jax.lax.rsqrt(var + _EPS) * gamma_ref[...]
    bb = beta_ref[...] - mean * a
    t = x * (0.5 * a) + 0.5 * bb
    return t + t * jnp.tanh(t)


def _fill_pad(pad_ref, y_hwc, H, W):
    C = pad_ref.shape[2]
    zrow = jnp.zeros((1, W + 2, C), pad_ref.dtype)
    zcol = jnp.zeros((H + 2, 1, C), pad_ref.dtype)
    pad_ref[0:1] = zrow
    pad_ref[H + 1:H + 2] = zrow
    pad_ref[:, 0:1, :] = zcol
    pad_ref[:, W + 1:W + 2, :] = zcol
    pad_ref[1:H + 1, 1:W + 1, :] = y_hwc.astype(pad_ref.dtype)


def _conv_from_pad(pad_ref, w_ref, H, W, C, Cout):
    acc = None
    for dx in range(3):
        xs = pad_ref[:, dx:dx + W, :].reshape((H + 2) * W, C)
        for dy in range(3):
            t = jnp.dot(xs[dy * W:dy * W + H * W], w_ref[3 * dy + dx],
                        preferred_element_type=jnp.float32)
            acc = t if acc is None else acc + t
    return acc


def _gnconv_body(x_ref, gamma_ref, beta_ref, grp_ref, w_ref, b_ref, o_ref,
                 pad_ref, *, H, W, C, Cout, inv_n, use_gn):
    xm = x_ref[0].reshape(H * W, C).astype(jnp.float32)
    y = _gn_swish(xm, grp_ref, gamma_ref, beta_ref, inv_n) if use_gn else xm
    _fill_pad(pad_ref, y.reshape(H, W, C), H, W)
    acc = _conv_from_pad(pad_ref, w_ref, H, W, C, Cout) + b_ref[...]
    o_ref[0] = acc.reshape(H, W, Cout)


def _res_body(x_ref, g0_ref, b0_ref, g1_ref, b1_ref, grp0_ref, grp1_ref,
              w0_ref, w1_ref, *rest, H, W, Cin, Cout, inv0, inv1, proj):
    if proj:
        wr_ref, o_ref, pad0_ref, pad1_ref = rest
    else:
        o_ref, pad0_ref, pad1_ref = rest
    xm = x_ref[0].reshape(H * W, Cin).astype(jnp.float32)
    y0 = _gn_swish(xm, grp0_ref, g0_ref, b0_ref, inv0)
    _fill_pad(pad0_ref, y0.reshape(H, W, Cin), H, W)
    h = _conv_from_pad(pad0_ref, w0_ref, H, W, Cin, Cout)
    y1 = _gn_swish(h, grp1_ref, g1_ref, b1_ref, inv1)
    _fill_pad(pad1_ref, y1.reshape(H, W, Cout), H, W)
    h2 = _conv_from_pad(pad1_ref, w1_ref, H, W, Cout, Cout)
    if proj:
        out = h2 + jnp.dot(h2.astype(_MMDT), wr_ref[...],
                           preferred_element_type=jnp.float32)
    else:
        out = h2 + xm
    o_ref[0] = out.reshape(H, W, Cout)


def _upconv_body(x_ref, w_ref, b_ref, o_ref, pad_ref, *, H, W, C):
    x = x_ref[0]
    xw = jnp.concatenate([x[:, :, None, :], x[:, :, None, :]],
                         axis=2).reshape(H, 2 * W, C)
    xh = jnp.concatenate([xw[:, None], xw[:, None]],
                         axis=1).reshape(2 * H, 2 * W, C)
    _fill_pad(pad_ref, xh, 2 * H, 2 * W)
    acc = _conv_from_pad(pad_ref, w_ref, 2 * H, 2 * W, C, C) + b_ref[...]
    o_ref[0] = acc.reshape(2 * H, 2 * W, C)


def _grp_matrix(C):
    gid = jnp.arange(C, dtype=jnp.int32) // (C // _GROUPS)
    return (gid[:, None] == gid[None, :]).astype(jnp.float32)


def _wmat3x3(w):
    Cout, Cin = w.shape[0], w.shape[1]
    return jnp.transpose(w, (2, 3, 1, 0)).reshape(9, Cin, Cout).astype(_MMDT)


def _bias_row(b, Cout):
    return (jnp.zeros((1, Cout), jnp.float32) if b is None
            else b.reshape(1, Cout).astype(jnp.float32))


_PAR = pltpu.CompilerParams(dimension_semantics=("parallel",))


def _gn_conv(x, gamma, beta, w, b, *, use_gn):
    B, H, W, C = x.shape
    Cout = w.shape[0]
    if use_gn:
        gam = gamma.reshape(1, C).astype(jnp.float32)
        bet = beta.reshape(1, C).astype(jnp.float32)
    else:
        gam = jnp.ones((1, C), jnp.float32)
        bet = jnp.zeros((1, C), jnp.float32)
    body = functools.partial(_gnconv_body, H=H, W=W, C=C, Cout=Cout,
                             inv_n=1.0 / float(H * W * (C // _GROUPS)),
                             use_gn=use_gn)
    return pl.pallas_call(
        body,
        out_shape=jax.ShapeDtypeStruct((B, H, W, Cout), jnp.float32),
        grid_spec=pltpu.PrefetchScalarGridSpec(
            num_scalar_prefetch=0,
            grid=(B,),
            in_specs=[
                pl.BlockSpec((1, H, W, C), lambda i: (i, 0, 0, 0)),
                pl.BlockSpec((1, C), lambda i: (0, 0)),
                pl.BlockSpec((1, C), lambda i: (0, 0)),
                pl.BlockSpec((C, C), lambda i: (0, 0)),
                pl.BlockSpec((9, C, Cout), lambda i: (0, 0, 0)),
                pl.BlockSpec((1, Cout), lambda i: (0, 0)),
            ],
            out_specs=pl.BlockSpec((1, H, W, Cout), lambda i: (i, 0, 0, 0)),
            scratch_shapes=[pltpu.VMEM((H + 2, W + 2, C), _MMDT)],
        ),
        compiler_params=_PAR,
    )(x, gam, bet, _grp_matrix(C), _wmat3x3(w), _bias_row(b, Cout))


def _res_block(x, g0, b0, w0, g1, b1, w1, wr):
    B, H, W, Cin = x.shape
    Cout = w0.shape[0]
    proj = wr is not None
    body = functools.partial(_res_body, H=H, W=W, Cin=Cin, Cout=Cout,
                             inv0=1.0 / float(H * W * (Cin // _GROUPS)),
                             inv1=1.0 / float(H * W * (Cout // _GROUPS)),
                             proj=proj)
    in_specs = [
        pl.BlockSpec((1, H, W, Cin), lambda i: (i, 0, 0, 0)),
        pl.BlockSpec((1, Cin), lambda i: (0, 0)),
        pl.BlockSpec((1, Cin), lambda i: (0, 0)),
        pl.BlockSpec((1, Cout), lambda i: (0, 0)),
        pl.BlockSpec((1, Cout), lambda i: (0, 0)),
        pl.BlockSpec((Cin, Cin), lambda i: (0, 0)),
        pl.BlockSpec((Cout, Cout), lambda i: (0, 0)),
        pl.BlockSpec((9, Cin, Cout), lambda i: (0, 0, 0)),
        pl.BlockSpec((9, Cout, Cout), lambda i: (0, 0, 0)),
    ]
    args = [x,
            g0.reshape(1, Cin).astype(jnp.float32),
            b0.reshape(1, Cin).astype(jnp.float32),
            g1.reshape(1, Cout).astype(jnp.float32),
            b1.reshape(1, Cout).astype(jnp.float32),
            _grp_matrix(Cin), _grp_matrix(Cout), _wmat3x3(w0), _wmat3x3(w1)]
    if proj:
        in_specs.append(pl.BlockSpec((Cout, Cout), lambda i: (0, 0)))
        args.append(jnp.transpose(wr.reshape(Cout, Cout)).astype(_MMDT))
    return pl.pallas_call(
        body,
        out_shape=jax.ShapeDtypeStruct((B, H, W, Cout), jnp.float32),
        grid_spec=pltpu.PrefetchScalarGridSpec(
            num_scalar_prefetch=0,
            grid=(B,),
            in_specs=in_specs,
            out_specs=pl.BlockSpec((1, H, W, Cout), lambda i: (i, 0, 0, 0)),
            scratch_shapes=[pltpu.VMEM((H + 2, W + 2, Cin), _MMDT),
                            pltpu.VMEM((H + 2, W + 2, Cout), _MMDT)],
        ),
        compiler_params=_PAR,
    )(*args)


def _up_conv(x, w, b):
    B, H, W, C = x.shape
    Cout = w.shape[0]
    body = functools.partial(_upconv_body, H=H, W=W, C=C)
    return pl.pallas_call(
        body,
        out_shape=jax.ShapeDtypeStruct((B, 2 * H, 2 * W, Cout), jnp.float32),
        grid_spec=pltpu.PrefetchScalarGridSpec(
            num_scalar_prefetch=0,
            grid=(B,),
            in_specs=[
                pl.BlockSpec((1, H, W, C), lambda i: (i, 0, 0, 0)),
                pl.BlockSpec((9, C, Cout), lambda i: (0, 0, 0)),
                pl.BlockSpec((1, Cout), lambda i: (0, 0)),
            ],
            out_specs=pl.BlockSpec((1, 2 * H, 2 * W, Cout),
                                   lambda i: (i, 0, 0, 0)),
            scratch_shapes=[pltpu.VMEM((2 * H + 2, 2 * W + 2, C), _MMDT)],
        ),
        compiler_params=_PAR,
    )(x, _wmat3x3(w), _bias_row(b, Cout))


def kernel(x, p00, p01, p02, p03, p04, p05, p06, p07, p08, p09, p10, p11,
           p12, p13, p14, p15, p16, p17, p18, p19, p20, p21, p22, p23, p24,
           p25, p26):
    h = jnp.transpose(x, (0, 2, 3, 1)).astype(jnp.float32)
    h = _gn_conv(h, None, None, p01, p00, use_gn=False)
    h = _res_block(h, p09, p08, p06, p11, p10, p07, None)
    h = _res_block(h, p15, p14, p12, p17, p16, p13, None)
    h = _up_conv(h, p19, p18)
    h = _res_block(h, p24, p23, p20, p26, p25, p21, p22)
    h = _gn_conv(h, p05, p04, p03, p02, use_gn=True)
    return jnp.transpose(h, (0, 3, 1, 2))
